# baseline (speedup 1.0000x reference)
import functools
import jax
import jax.numpy as jnp
from jax import lax
from jax.experimental import pallas as pl
from jax.experimental.pallas import tpu as pltpu
from jax.experimental.pallas import tpu_sc as plsc

NUM_EMBEDDINGS = 1000000
EMBEDDING_DIM = 64
SEQ_LEN = 200
BATCH = 4096

_NC, _NS = 2, 16
_NW = _NC * _NS
_V2 = NUM_EMBEDDINGS // 2
_NB1 = (NUM_EMBEDDINGS + 127) // 128
_S1 = (_NB1 // _NW + 4) // 4
_CB = 128
_NUPW = SEQ_LEN * (BATCH // _CB) // _NW

_params = pltpu.CompilerParams(
    use_tc_tiling_on_sc=True, needs_layout_passes=False)


def _iota16():
    return lax.iota(jnp.int32, 16)


def _make_retile():
    mesh = plsc.VectorSubcoreMesh(core_axis_name="c", subcore_axis_name="s")

    @functools.partial(
        pl.kernel,
        mesh=mesh,
        out_type=jax.ShapeDtypeStruct((_V2, 128), jnp.float32),
        compiler_params=_params,
        scratch_types=[
            pltpu.VMEM((4, 64, 128), jnp.float32),
            pltpu.VMEM((2, 64, 128), jnp.float32),
            pltpu.SemaphoreType.DMA,
            pltpu.SemaphoreType.DMA,
            pltpu.SemaphoreType.DMA,
            pltpu.SemaphoreType.DMA,
            pltpu.SemaphoreType.DMA,
            pltpu.SemaphoreType.DMA,
        ],
    )
    def retile_kernel(wt_hbm, t2_hbm, in_v, out_v,
                      gi0, gi1, gi2, gi3, go0, go1):
        wid = lax.axis_index("s") * _NC + lax.axis_index("c")
        rows16 = _iota16()
        gis = (gi0, gi1, gi2, gi3)
        gos = (go0, go1)

        def start_in(j, bi):
            pltpu.async_copy(
                wt_hbm.at[:, pl.ds(bi * 128, 128)], in_v.at[j], gis[j])

        def wait_in(j):
            pltpu.make_async_copy(
                wt_hbm.at[:, pl.ds(0, 128)], in_v.at[j], gis[j]).wait()

        def start_out(ob, bi):
            @pl.when(bi < _NB1 - 1)
            def _():
                pltpu.async_copy(
                    out_v.at[ob], t2_hbm.at[pl.ds(bi * 64, 64), :], gos[ob])

            @pl.when(bi == _NB1 - 1)
            def _():
                pltpu.async_copy(
                    out_v.at[ob, pl.ds(0, 32)],
                    t2_hbm.at[pl.ds(bi * 64, 32), :], gos[ob])

        def wait_out(ob, bi):
            @pl.when(bi < _NB1 - 1)
            def _():
                pltpu.make_async_copy(
                    out_v.at[ob], t2_hbm.at[pl.ds(0, 64), :], gos[ob]).wait()

            @pl.when(bi == _NB1 - 1)
            def _():
                pltpu.make_async_copy(
                    out_v.at[ob, pl.ds(0, 32)],
                    t2_hbm.at[pl.ds(0, 32), :], gos[ob]).wait()

        def transpose(j, ob):
            @plsc.parallel_loop(0, 64, 1, unroll=8)
            def _t(p):
                for h in range(2):
                    col = jnp.full((16,), 2, jnp.int32) * p + h
                    for g in range(4):
                        vec = plsc.load_gather(
                            in_v.at[j], [g * 16 + rows16, col])
                        out_v[ob, p, pl.ds(h * 64 + g * 16, 16)] = vec

        for j in range(4):
            start_in(j, wid + j * _NW)

        def body(s, carry):
            for j in range(4):
                m = 4 * s + j
                bi = wid + m * _NW

                @pl.when(bi < _NB1)
                def _sec(j=j, m=m, bi=bi):
                    wait_in(j)

                    @pl.when(m >= 2)
                    def _():
                        wait_out(j % 2, bi - 2 * _NW)

                    transpose(j, j % 2)
                    start_out(j % 2, bi)

                    @pl.when(bi + 4 * _NW < _NB1)
                    def _():
                        start_in(j, bi + 4 * _NW)

            return carry

        lax.fori_loop(0, _S1, body, 0)

        nblk = (_NB1 - 1 - wid) // _NW + 1
        last_even = ((nblk - 1) // 2) * 2
        last_odd = ((nblk - 2) // 2) * 2 + 1
        wait_out(0, wid + last_even * _NW)
        wait_out(1, wid + last_odd * _NW)

    return retile_kernel


def _make_gather():
    mesh = plsc.VectorSubcoreMesh(core_axis_name="c", subcore_axis_name="s")

    @functools.partial(
        pl.kernel,
        mesh=mesh,
        out_type=jax.ShapeDtypeStruct((SEQ_LEN, EMBEDDING_DIM, BATCH),
                                      jnp.float32),
        compiler_params=_params,
        scratch_types=[
            pltpu.VMEM((4, _CB), jnp.int32),
            pltpu.VMEM((_CB,), jnp.int32),
            pltpu.VMEM((_CB,), jnp.int32),
            pltpu.VMEM((_CB,), jnp.int32),
            pltpu.VMEM((_CB,), jnp.int32),
            pltpu.VMEM((4, _CB, 128), jnp.float32),
            pltpu.VMEM((2, EMBEDDING_DIM, _CB), jnp.float32),
            pltpu.SemaphoreType.DMA,
            pltpu.SemaphoreType.DMA,
            pltpu.SemaphoreType.DMA,
            pltpu.SemaphoreType.DMA,
            pltpu.SemaphoreType.DMA,
            pltpu.SemaphoreType.DMA,
            pltpu.SemaphoreType.DMA,
            pltpu.SemaphoreType.DMA,
            pltpu.SemaphoreType.DMA,
            pltpu.SemaphoreType.DMA,
        ],
    )
    def gather_kernel(t2_hbm, idx_hbm, out_hbm, idx_v, px0, px1, px2, px3,
                      g_v, o_v, gg0, gg1, gg2, gg3, gx0, gx1, gx2, gx3,
                      go0, go1):
        wid = lax.axis_index("s") * _NC + lax.axis_index("c")
        rows16 = _iota16()
        ggs = (gg0, gg1, gg2, gg3)
        gxs = (gx0, gx1, gx2, gx3)
        gos = (go0, go1)
        pxs = (px0, px1, px2, px3)
        nbt = BATCH // _CB

        def uaddr(m):
            u = wid + m * _NW
            return u // nbt, (u % nbt) * _CB

        def start_idx(j, m):
            t, b0 = uaddr(m)
            pltpu.async_copy(
                idx_hbm.at[t, pl.ds(b0, _CB)], idx_v.at[j], gxs[j])

        def wait_idx_make_pidx(j):
            pltpu.make_async_copy(
                idx_hbm.at[0, pl.ds(0, _CB)], idx_v.at[j], gxs[j]).wait()
            for ig in range(_CB // 16):
                pxs[j][pl.ds(ig * 16, 16)] = idx_v[j, pl.ds(ig * 16, 16)] >> 1

        def start_gather(j):
            pltpu.async_copy(t2_hbm.at[pxs[j]], g_v.at[j], ggs[j])

        def wait_gather(j):
            pltpu.make_async_copy(
                t2_hbm.at[pxs[j]], g_v.at[j], ggs[j]).wait()

        def start_out(ob, m):
            t, b0 = uaddr(m)
            pltpu.async_copy(
                o_v.at[ob], out_hbm.at[t, :, pl.ds(b0, _CB)], gos[ob])

        def wait_out(ob):
            pltpu.make_async_copy(
                o_v.at[ob], out_hbm.at[0, :, pl.ds(0, _CB)], gos[ob]).wait()

        def transpose(j, ob):
            @plsc.parallel_loop(0, _CB // 16, 1, unroll=4)
            def _t(ig):
                halfv = (idx_v[j, pl.ds(ig * 16, 16)] & 1) * 64
                rows = ig * 16 + rows16
                for d in range(EMBEDDING_DIM):
                    vec = plsc.load_gather(g_v.at[j], [rows, halfv + d])
                    o_v[ob, d, pl.ds(ig * 16, 16)] = vec

        for j in range(4):
            start_idx(j, j)
        wait_idx_make_pidx(0)
        start_gather(0)
        wait_idx_make_pidx(1)
        start_gather(1)

        def body(s, carry):
            for j in range(4):
                m = 4 * s + j
                jn = (j + 2) % 4

                wait_gather(j)

                @pl.when(m >= 2)
                def _(j=j):
                    wait_out(j % 2)

                transpose(j, j % 2)
                start_out(j % 2, m)

                @pl.when(m + 4 < _NUPW)
                def _(j=j, m=m):
                    start_idx(j, m + 4)

                @pl.when(m + 2 < _NUPW)
                def _(jn=jn, m=m):
                    wait_idx_make_pidx(jn)
                    start_gather(jn)

            return carry

        lax.fori_loop(0, _NUPW // 4, body, 0)
        wait_out(0)
        wait_out(1)

    return gather_kernel


_retile = _make_retile()
_gather = _make_gather()


def kernel(input, weight):
    t2 = _retile(weight.T)
    out_t = _gather(t2, input)
    return jnp.transpose(out_t, (0, 2, 1))


# speedup vs baseline: 1.3962x; 1.3962x over previous
import functools
import jax
import jax.numpy as jnp
from jax import lax
from jax.experimental import pallas as pl
from jax.experimental.pallas import tpu as pltpu
from jax.experimental.pallas import tpu_sc as plsc

NUM_EMBEDDINGS = 1000000
EMBEDDING_DIM = 64
SEQ_LEN = 200
BATCH = 4096

_B = SEQ_LEN * BATCH
_NC, _NS = 2, 16
_NW = _NC * _NS
_BPW = _B // _NW
_C = 800
_NCHUNK = _BPW // _C


def _make_gather():
    mesh = plsc.VectorSubcoreMesh(core_axis_name="c", subcore_axis_name="s")

    @functools.partial(
        pl.kernel,
        mesh=mesh,
        out_type=jax.ShapeDtypeStruct((_B, EMBEDDING_DIM), jnp.float32),
        compiler_params=pltpu.CompilerParams(use_tc_tiling_on_sc=False),
        scratch_types=[
            pltpu.VMEM((2, _C), jnp.int32),
            pltpu.VMEM((2, _C, EMBEDDING_DIM), jnp.float32),
            pltpu.SemaphoreType.DMA,
            pltpu.SemaphoreType.DMA,
            pltpu.SemaphoreType.DMA,
            pltpu.SemaphoreType.DMA,
        ],
    )
    def gather_kernel(table_hbm, idx_hbm, out_hbm, idx_v, rows_v, g0, g1,
                      s0, s1):
        wid = lax.axis_index("s") * _NC + lax.axis_index("c")
        base = wid * _BPW
        gsem = (g0, g1)
        ssem = (s0, s1)

        def load_idx_and_gather(b, off):
            pltpu.sync_copy(idx_hbm.at[pl.ds(off, _C)], idx_v.at[b])
            pltpu.async_copy(table_hbm.at[idx_v.at[b]], rows_v.at[b], gsem[b])

        def wait_gather(b):
            pltpu.make_async_copy(
                table_hbm.at[idx_v.at[b]], rows_v.at[b], gsem[b]).wait()

        def start_store(b, off):
            pltpu.async_copy(rows_v.at[b], out_hbm.at[pl.ds(off, _C)], ssem[b])

        def wait_store(b):
            pltpu.make_async_copy(
                rows_v.at[b], out_hbm.at[pl.ds(base, _C)], ssem[b]).wait()

        load_idx_and_gather(0, base)

        def body(s, carry):
            off0 = base + (2 * s) * _C
            off1 = off0 + _C

            wait_gather(0)

            @pl.when(s > 0)
            def _():
                wait_store(1)

            load_idx_and_gather(1, off1)
            start_store(0, off0)

            wait_gather(1)
            wait_store(0)

            @pl.when(s + 1 < _NCHUNK // 2)
            def _():
                load_idx_and_gather(0, off1 + _C)

            start_store(1, off1)
            return carry

        lax.fori_loop(0, _NCHUNK // 2, body, 0)
        wait_store(1)

    return gather_kernel


_gather = _make_gather()


def kernel(input, weight):
    flat_idx = input.reshape(-1)
    rows = _gather(weight, flat_idx)
    return rows.reshape(input.shape + (weight.shape[1],))
